# Initial kernel scaffold; baseline (speedup 1.0000x reference)
import jax
import jax.numpy as jnp
from jax.experimental import pallas as pl


def kernel(hidden_states, W_router, W_gate, W_up, W_down):
    raise NotImplementedError("write your pallas kernel here")


# speedup vs baseline: 3.6764x; 3.6764x over previous
import jax
import jax.numpy as jnp
from jax.experimental import pallas as pl
from jax.experimental.pallas import tpu as pltpu

_E = 64
_TOPK = 2


def _moe_dense_body(x_ref, wr_ref, wg_ref, wu_ref, wd_ref, out_ref, dw_ref):
    e = pl.program_id(0)
    T = x_ref.shape[0]

    @pl.when(e == 0)
    def _router():
        x = x_ref[...]
        logits = jax.lax.dot_general(
            x, wr_ref[...], (((1,), (1,)), ((), ())),
            preferred_element_type=jnp.float32)
        m = jnp.max(logits, axis=1, keepdims=True)
        p = jnp.exp(logits - m)
        probs = p / jnp.sum(p, axis=1, keepdims=True)
        lane = jax.lax.broadcasted_iota(jnp.int32, probs.shape, 1)
        m1 = jnp.max(probs, axis=1, keepdims=True)
        i1 = jnp.min(jnp.where(probs == m1, lane, _E), axis=1, keepdims=True)
        probs2 = jnp.where(lane == i1, -jnp.inf, probs)
        m2 = jnp.max(probs2, axis=1, keepdims=True)
        i2 = jnp.min(jnp.where(probs2 == m2, lane, _E), axis=1, keepdims=True)
        s = m1 + m2
        dw_ref[...] = (jnp.where(lane == i1, m1 / s, 0.0)
                       + jnp.where(lane == i2, m2 / s, 0.0))

    x = x_ref[...]
    g = jax.lax.dot_general(x, wg_ref[0], (((1,), (1,)), ((), ())),
                            preferred_element_type=jnp.float32)
    u = jax.lax.dot_general(x, wu_ref[0], (((1,), (1,)), ((), ())),
                            preferred_element_type=jnp.float32)
    h = (g * jax.nn.sigmoid(g)) * u
    o = jax.lax.dot_general(h, wd_ref[0], (((1,), (1,)), ((), ())),
                            preferred_element_type=jnp.float32)
    lane = jax.lax.broadcasted_iota(jnp.int32, (T, _E), 1)
    we = jnp.sum(jnp.where(lane == e, dw_ref[...], 0.0), axis=1, keepdims=True)
    contrib = we * o

    @pl.when(e == 0)
    def _init():
        out_ref[...] = contrib

    @pl.when(e > 0)
    def _acc():
        out_ref[...] += contrib


def kernel(hidden_states, W_router, W_gate, W_up, W_down):
    b, s, d = hidden_states.shape
    x = hidden_states.reshape(-1, d)
    T = x.shape[0]
    E, FF = W_gate.shape[0], W_gate.shape[1]
    out = pl.pallas_call(
        _moe_dense_body,
        grid=(E,),
        in_specs=[
            pl.BlockSpec((T, d), lambda e: (0, 0)),
            pl.BlockSpec((E, d), lambda e: (0, 0)),
            pl.BlockSpec((1, FF, d), lambda e: (e, 0, 0)),
            pl.BlockSpec((1, FF, d), lambda e: (e, 0, 0)),
            pl.BlockSpec((1, d, FF), lambda e: (e, 0, 0)),
        ],
        out_specs=pl.BlockSpec((T, d), lambda e: (0, 0)),
        out_shape=jax.ShapeDtypeStruct((T, d), jnp.float32),
        scratch_shapes=[pltpu.VMEM((T, E), jnp.float32)],
    )(x, W_router, W_gate, W_up, W_down)
    return out.reshape(b, s, d)
